# Initial kernel scaffold; baseline (speedup 1.0000x reference)
import jax
import jax.numpy as jnp
from jax.experimental import pallas as pl


def kernel(x1, edge_index1, x2, edge_index2, W1, b1, W2, b2):
    raise NotImplementedError("write your pallas kernel here")


# speedup vs baseline: 82.8702x; 82.8702x over previous
import functools

import jax
import jax.numpy as jnp
from jax import lax
from jax.experimental import pallas as pl
from jax.experimental.pallas import tpu as pltpu
from jax.experimental.pallas import tpu_sc as plsc

NC = 2
NS = 16
NW = NC * NS
LANES = 16
B = 128
CB = 200


def _sc_mesh():
    return plsc.VectorSubcoreMesh(core_axis_name="c", subcore_axis_name="s")


def _zero_slice(zbuf, n):
    def body(i, _):
        zbuf[pl.ds(i * LANES, LANES)] = jnp.zeros((LANES,), jnp.float32)
        return _
    lax.fori_loop(0, n // LANES, body, None)


def _degree_kernel(nb, nch, npad, dst_hbm, deg_out, dst_buf, ones, zbuf, deg_sp):
    s = lax.axis_index("s")
    c = lax.axis_index("c")
    wid = c * NS + s
    sl = npad // NS
    _zero_slice(zbuf, sl)
    pltpu.sync_copy(zbuf, deg_sp.at[pl.ds(s * sl, sl)])
    for i in range(B // LANES):
        ones[pl.ds(i * LANES, LANES)] = jnp.ones((LANES,), jnp.float32)
    plsc.subcore_barrier()
    for t in range((nch + NW - 1) // NW):
        ch = wid + NW * t
        @pl.when(ch < nch)
        def _():
            pltpu.sync_copy(dst_hbm.at[pl.ds(ch * CB, CB)], dst_buf)
            def body(j, _):
                pltpu.sync_copy(ones, deg_sp.at[dst_buf.at[j]], add=True)
                return _
            lax.fori_loop(0, CB, body, None)
    plsc.subcore_barrier()
    pltpu.sync_copy(deg_sp.at[pl.ds(s * sl, sl)],
                    deg_out.at[c, pl.ds(s * sl, sl)])


def _prop_kernel(nb, nch, npad, src_hbm, dst_hbm, ua_hbm, ub_hbm,
                 acca_out, accb_out,
                 src_buf, dst_buf, ga, gb, zbuf, ua_sp, ub_sp, aa_sp, ab_sp):
    s = lax.axis_index("s")
    c = lax.axis_index("c")
    wid = c * NS + s
    sl = npad // NS
    _zero_slice(zbuf, sl)
    pltpu.sync_copy(zbuf, aa_sp.at[pl.ds(s * sl, sl)])
    pltpu.sync_copy(zbuf, ab_sp.at[pl.ds(s * sl, sl)])
    pltpu.sync_copy(ua_hbm.at[pl.ds(s * sl, sl)], ua_sp.at[pl.ds(s * sl, sl)])
    pltpu.sync_copy(ub_hbm.at[pl.ds(s * sl, sl)], ub_sp.at[pl.ds(s * sl, sl)])
    plsc.subcore_barrier()
    for t in range((nch + NW - 1) // NW):
        ch = wid + NW * t
        @pl.when(ch < nch)
        def _():
            pltpu.sync_copy(src_hbm.at[pl.ds(ch * CB, CB)], src_buf)
            pltpu.sync_copy(dst_hbm.at[pl.ds(ch * CB, CB)], dst_buf)
            def body(j, _):
                pltpu.sync_copy(ua_sp.at[src_buf.at[j]], ga)
                pltpu.sync_copy(ub_sp.at[src_buf.at[j]], gb)
                pltpu.sync_copy(ga, aa_sp.at[dst_buf.at[j]], add=True)
                pltpu.sync_copy(gb, ab_sp.at[dst_buf.at[j]], add=True)
                return _
            lax.fori_loop(0, CB, body, None)
    plsc.subcore_barrier()
    pltpu.sync_copy(aa_sp.at[pl.ds(s * sl, sl)], acca_out.at[c, pl.ds(s * sl, sl)])
    pltpu.sync_copy(ab_sp.at[pl.ds(s * sl, sl)], accb_out.at[c, pl.ds(s * sl, sl)])


def _make_degree(nb, npad):
    nch = nb // CB
    return pl.kernel(
        functools.partial(_degree_kernel, nb, nch, npad),
        out_type=jax.ShapeDtypeStruct((NC, npad), jnp.float32),
        mesh=_sc_mesh(),
        scratch_types=[
            pltpu.VMEM((CB, B), jnp.int32),
            pltpu.VMEM((B,), jnp.float32),
            pltpu.VMEM((npad // NS,), jnp.float32),
            pltpu.MemorySpace.VMEM_SHARED((npad,), jnp.float32),
        ],
    )


def _make_prop(nb, npad):
    nch = nb // CB
    return pl.kernel(
        functools.partial(_prop_kernel, nb, nch, npad),
        out_type=(jax.ShapeDtypeStruct((NC, npad), jnp.float32),
                  jax.ShapeDtypeStruct((NC, npad), jnp.float32)),
        mesh=_sc_mesh(),
        scratch_types=[
            pltpu.VMEM((CB, B), jnp.int32),
            pltpu.VMEM((CB, B), jnp.int32),
            pltpu.VMEM((B,), jnp.float32),
            pltpu.VMEM((B,), jnp.float32),
            pltpu.VMEM((npad // NS,), jnp.float32),
            pltpu.MemorySpace.VMEM_SHARED((npad,), jnp.float32),
            pltpu.MemorySpace.VMEM_SHARED((npad,), jnp.float32),
            pltpu.MemorySpace.VMEM_SHARED((npad,), jnp.float32),
            pltpu.MemorySpace.VMEM_SHARED((npad,), jnp.float32),
        ],
    )


def _tc_a_body(degp_ref, xa_ref, xb_ref, dinv_ref, ua_ref, ub_ref):
    deg = degp_ref[0] + degp_ref[1] + 1.0
    dinv = lax.rsqrt(deg)
    dinv_ref[...] = dinv
    ua_ref[...] = dinv * xa_ref[...]
    ub_ref[...] = dinv * xb_ref[...]


def _tc_b_body(acca_ref, accb_ref, dinv_ref, ua_ref, ub_ref,
               w1_ref, b1_ref, w2_ref, u2a_ref, u2b_ref):
    dinv = dinv_ref[...]
    pre0 = dinv * (acca_ref[0] + acca_ref[1] + ua_ref[...])
    pre1 = dinv * (accb_ref[0] + accb_ref[1] + ub_ref[...])
    y0 = jnp.zeros_like(pre0)
    y1 = jnp.zeros_like(pre0)
    for j in range(16):
        h = jnp.maximum(pre0 * w1_ref[0, j] + pre1 * w1_ref[1, j] + b1_ref[j], 0.0)
        y0 = y0 + h * w2_ref[j, 0]
        y1 = y1 + h * w2_ref[j, 1]
    u2a_ref[...] = dinv * y0
    u2b_ref[...] = dinv * y1


def _tc_c_body(acca_ref, accb_ref, dinv_ref, u2a_ref, u2b_ref, b2_ref,
               o0_ref, o1_ref):
    dinv = dinv_ref[...]
    z0 = dinv * (acca_ref[0] + acca_ref[1] + u2a_ref[...]) + b2_ref[0]
    z1 = dinv * (accb_ref[0] + accb_ref[1] + u2b_ref[...]) + b2_ref[1]
    m = jnp.maximum(z0, z1)
    ls = jnp.log(jnp.exp(z0 - m) + jnp.exp(z1 - m))
    o0_ref[...] = z0 - m - ls
    o1_ref[...] = z1 - m - ls


def _vspec():
    return pl.BlockSpec(memory_space=pltpu.MemorySpace.VMEM)


def _sspec():
    return pl.BlockSpec(memory_space=pltpu.MemorySpace.SMEM)


def _plane(shape):
    return jax.ShapeDtypeStruct(shape, jnp.float32)


def kernel(x1, edge_index1, x2, edge_index2, W1, b1, W2, b2):
    del x2, edge_index2
    n = x1.shape[0]
    e = edge_index1.shape[1]
    npad = ((n + 1023) // 1024) * 1024
    assert e % (B * CB) == 0
    nb = e // B
    r = npad // 128

    src = edge_index1[0].reshape(nb, B)
    dst = edge_index1[1].reshape(nb, B)
    xp = jnp.pad(x1, ((0, npad - n), (0, 0)))
    xa = xp[:, 0]
    xb = xp[:, 1]

    degp = _make_degree(nb, npad)(dst)

    dinv, ua, ub = pl.pallas_call(
        _tc_a_body,
        out_shape=[_plane((r, 128))] * 3,
        in_specs=[_vspec()] * 3,
        out_specs=[_vspec()] * 3,
    )(degp.reshape(NC, r, 128), xa.reshape(r, 128), xb.reshape(r, 128))

    prop = _make_prop(nb, npad)
    acc1a, acc1b = prop(src, dst, ua.reshape(npad), ub.reshape(npad))

    u2a, u2b = pl.pallas_call(
        _tc_b_body,
        out_shape=[_plane((r, 128))] * 2,
        in_specs=[_vspec()] * 5 + [_sspec()] * 3,
        out_specs=[_vspec()] * 2,
    )(acc1a.reshape(NC, r, 128), acc1b.reshape(NC, r, 128), dinv,
      ua, ub, W1, b1, W2)

    acc2a, acc2b = prop(src, dst, u2a.reshape(npad), u2b.reshape(npad))

    o0, o1 = pl.pallas_call(
        _tc_c_body,
        out_shape=[_plane((r, 128))] * 2,
        in_specs=[_vspec()] * 5 + [_sspec()],
        out_specs=[_vspec()] * 2,
    )(acc2a.reshape(NC, r, 128), acc2b.reshape(NC, r, 128), dinv,
      u2a, u2b, b2)

    return jnp.stack([o0.reshape(npad), o1.reshape(npad)], axis=1)[:n]


# speedup vs baseline: 177.4263x; 2.1410x over previous
import functools

import jax
import jax.numpy as jnp
from jax import lax
from jax.experimental import pallas as pl
from jax.experimental.pallas import tpu as pltpu
from jax.experimental.pallas import tpu_sc as plsc

NC = 2
NS = 16
NW = NC * NS
LANES = 16
B = 128
CB = 40


def _sc_mesh():
    return plsc.VectorSubcoreMesh(core_axis_name="c", subcore_axis_name="s")


def _zero_slice(zbuf, n):
    def body(i, _):
        zbuf[pl.ds(i * LANES, LANES)] = jnp.zeros((LANES,), jnp.float32)
        return _
    lax.fori_loop(0, n // LANES, body, None)


def _degree_kernel(nch, npad, dst_hbm, deg_out, dst_bufs, ones, zbuf, deg_sp, sems):
    s = lax.axis_index("s")
    c = lax.axis_index("c")
    wid = c * NS + s
    sl = npad // NS
    _zero_slice(zbuf, sl)
    pltpu.sync_copy(zbuf, deg_sp.at[pl.ds(s * sl, sl)])
    for i in range(B // LANES):
        ones[pl.ds(i * LANES, LANES)] = jnp.ones((LANES,), jnp.float32)
    plsc.subcore_barrier()

    nt = (nch + NW - 1) // NW

    def fire(dbuf, sem):
        def body(j, _):
            pltpu.async_copy(ones, deg_sp.at[dbuf.at[j]], sem, add=True)
            return _
        lax.fori_loop(0, CB, body, None)

    def drain(dbuf, sem):
        def body(j, _):
            pltpu.make_async_copy(ones, deg_sp.at[dbuf.at[j]], sem).wait()
            return _
        lax.fori_loop(0, CB, body, None)

    for t in range(nt + 1):
        p = t % 2
        q = (t - 1) % 2
        if t >= 1 and t - 1 < nt:
            ch1 = wid + NW * (t - 1)
            @pl.when(ch1 < nch)
            def _():
                drain(dst_bufs[q], sems[q])
        if t < nt:
            ch = wid + NW * t
            @pl.when(ch < nch)
            def _():
                pltpu.sync_copy(dst_hbm.at[pl.ds(ch * CB, CB)], dst_bufs[p])
                fire(dst_bufs[p], sems[p])

    plsc.subcore_barrier()
    pltpu.sync_copy(deg_sp.at[pl.ds(s * sl, sl)],
                    deg_out.at[c, pl.ds(s * sl, sl)])


def _prop_kernel(nch, npad, src_hbm, dst_hbm, ua_hbm, ub_hbm, acca_out, accb_out,
                 src_bufs, dst_bufs, ga_bufs, gb_bufs, zbuf,
                 ua_sp, ub_sp, aa_sp, ab_sp, gsems, ssems):
    s = lax.axis_index("s")
    c = lax.axis_index("c")
    wid = c * NS + s
    sl = npad // NS
    _zero_slice(zbuf, sl)
    pltpu.sync_copy(zbuf, aa_sp.at[pl.ds(s * sl, sl)])
    pltpu.sync_copy(zbuf, ab_sp.at[pl.ds(s * sl, sl)])
    pltpu.sync_copy(ua_hbm.at[pl.ds(s * sl, sl)], ua_sp.at[pl.ds(s * sl, sl)])
    pltpu.sync_copy(ub_hbm.at[pl.ds(s * sl, sl)], ub_sp.at[pl.ds(s * sl, sl)])
    plsc.subcore_barrier()

    nt = (nch + NW - 1) // NW

    def fire_gathers(sbuf, ga, gb, sem):
        def body(j, _):
            pltpu.async_copy(ua_sp.at[sbuf.at[j]], ga.at[j], sem)
            pltpu.async_copy(ub_sp.at[sbuf.at[j]], gb.at[j], sem)
            return _
        lax.fori_loop(0, CB, body, None)

    def drain_gathers(sbuf, ga, gb, sem):
        def body(j, _):
            pltpu.make_async_copy(ua_sp.at[sbuf.at[j]], ga.at[j], sem).wait()
            pltpu.make_async_copy(ub_sp.at[sbuf.at[j]], gb.at[j], sem).wait()
            return _
        lax.fori_loop(0, CB, body, None)

    def fire_scatters(dbuf, ga, gb, sem):
        def body(j, _):
            pltpu.async_copy(ga.at[j], aa_sp.at[dbuf.at[j]], sem, add=True)
            pltpu.async_copy(gb.at[j], ab_sp.at[dbuf.at[j]], sem, add=True)
            return _
        lax.fori_loop(0, CB, body, None)

    def drain_scatters(dbuf, ga, gb, sem):
        def body(j, _):
            pltpu.make_async_copy(ga.at[j], aa_sp.at[dbuf.at[j]], sem).wait()
            pltpu.make_async_copy(gb.at[j], ab_sp.at[dbuf.at[j]], sem).wait()
            return _
        lax.fori_loop(0, CB, body, None)

    for t in range(nt + 2):
        p = t % 2
        q = (t - 1) % 2
        if t >= 2 and t - 2 < nt:
            ch2 = wid + NW * (t - 2)
            @pl.when(ch2 < nch)
            def _():
                drain_scatters(dst_bufs[p], ga_bufs[p], gb_bufs[p], ssems[p])
        if t < nt:
            ch = wid + NW * t
            @pl.when(ch < nch)
            def _():
                pltpu.sync_copy(src_hbm.at[pl.ds(ch * CB, CB)], src_bufs[p])
                pltpu.sync_copy(dst_hbm.at[pl.ds(ch * CB, CB)], dst_bufs[p])
                fire_gathers(src_bufs[p], ga_bufs[p], gb_bufs[p], gsems[p])
        if 1 <= t <= nt:
            ch1 = wid + NW * (t - 1)
            @pl.when(ch1 < nch)
            def _():
                drain_gathers(src_bufs[q], ga_bufs[q], gb_bufs[q], gsems[q])
                fire_scatters(dst_bufs[q], ga_bufs[q], gb_bufs[q], ssems[q])

    plsc.subcore_barrier()
    pltpu.sync_copy(aa_sp.at[pl.ds(s * sl, sl)], acca_out.at[c, pl.ds(s * sl, sl)])
    pltpu.sync_copy(ab_sp.at[pl.ds(s * sl, sl)], accb_out.at[c, pl.ds(s * sl, sl)])


def _make_degree(nb, npad):
    nch = nb // CB
    return pl.kernel(
        functools.partial(_degree_kernel, nch, npad),
        out_type=jax.ShapeDtypeStruct((NC, npad), jnp.float32),
        mesh=_sc_mesh(),
        scratch_types=[
            [pltpu.VMEM((CB, B), jnp.int32)] * 2,
            pltpu.VMEM((B,), jnp.float32),
            pltpu.VMEM((npad // NS,), jnp.float32),
            pltpu.MemorySpace.VMEM_SHARED((npad,), jnp.float32),
            [pltpu.SemaphoreType.DMA] * 2,
        ],
    )


def _make_prop(nb, npad):
    nch = nb // CB
    return pl.kernel(
        functools.partial(_prop_kernel, nch, npad),
        out_type=(jax.ShapeDtypeStruct((NC, npad), jnp.float32),
                  jax.ShapeDtypeStruct((NC, npad), jnp.float32)),
        mesh=_sc_mesh(),
        scratch_types=[
            [pltpu.VMEM((CB, B), jnp.int32)] * 2,
            [pltpu.VMEM((CB, B), jnp.int32)] * 2,
            [pltpu.VMEM((CB, B), jnp.float32)] * 2,
            [pltpu.VMEM((CB, B), jnp.float32)] * 2,
            pltpu.VMEM((npad // NS,), jnp.float32),
            pltpu.MemorySpace.VMEM_SHARED((npad,), jnp.float32),
            pltpu.MemorySpace.VMEM_SHARED((npad,), jnp.float32),
            pltpu.MemorySpace.VMEM_SHARED((npad,), jnp.float32),
            pltpu.MemorySpace.VMEM_SHARED((npad,), jnp.float32),
            [pltpu.SemaphoreType.DMA] * 2,
            [pltpu.SemaphoreType.DMA] * 2,
        ],
    )


def _tc_a_body(degp_ref, xa_ref, xb_ref, dinv_ref, ua_ref, ub_ref):
    deg = degp_ref[0] + degp_ref[1] + 1.0
    dinv = lax.rsqrt(deg)
    dinv_ref[...] = dinv
    ua_ref[...] = dinv * xa_ref[...]
    ub_ref[...] = dinv * xb_ref[...]


def _tc_b_body(acca_ref, accb_ref, dinv_ref, ua_ref, ub_ref,
               w1_ref, b1_ref, w2_ref, u2a_ref, u2b_ref):
    dinv = dinv_ref[...]
    pre0 = dinv * (acca_ref[0] + acca_ref[1] + ua_ref[...])
    pre1 = dinv * (accb_ref[0] + accb_ref[1] + ub_ref[...])
    y0 = jnp.zeros_like(pre0)
    y1 = jnp.zeros_like(pre0)
    for j in range(16):
        h = jnp.maximum(pre0 * w1_ref[0, j] + pre1 * w1_ref[1, j] + b1_ref[j], 0.0)
        y0 = y0 + h * w2_ref[j, 0]
        y1 = y1 + h * w2_ref[j, 1]
    u2a_ref[...] = dinv * y0
    u2b_ref[...] = dinv * y1


def _tc_c_body(acca_ref, accb_ref, dinv_ref, u2a_ref, u2b_ref, b2_ref,
               o0_ref, o1_ref):
    dinv = dinv_ref[...]
    z0 = dinv * (acca_ref[0] + acca_ref[1] + u2a_ref[...]) + b2_ref[0]
    z1 = dinv * (accb_ref[0] + accb_ref[1] + u2b_ref[...]) + b2_ref[1]
    m = jnp.maximum(z0, z1)
    ls = jnp.log(jnp.exp(z0 - m) + jnp.exp(z1 - m))
    o0_ref[...] = z0 - m - ls
    o1_ref[...] = z1 - m - ls


def _vspec():
    return pl.BlockSpec(memory_space=pltpu.MemorySpace.VMEM)


def _sspec():
    return pl.BlockSpec(memory_space=pltpu.MemorySpace.SMEM)


def _plane(shape):
    return jax.ShapeDtypeStruct(shape, jnp.float32)


def kernel(x1, edge_index1, x2, edge_index2, W1, b1, W2, b2):
    del x2, edge_index2
    n = x1.shape[0]
    e = edge_index1.shape[1]
    npad = ((n + 1023) // 1024) * 1024
    assert e % (B * CB) == 0
    nb = e // B
    r = npad // 128

    src = edge_index1[0].reshape(nb, B)
    dst = edge_index1[1].reshape(nb, B)
    xp = jnp.pad(x1, ((0, npad - n), (0, 0)))
    xa = xp[:, 0]
    xb = xp[:, 1]

    degp = _make_degree(nb, npad)(dst)

    dinv, ua, ub = pl.pallas_call(
        _tc_a_body,
        out_shape=[_plane((r, 128))] * 3,
        in_specs=[_vspec()] * 3,
        out_specs=[_vspec()] * 3,
    )(degp.reshape(NC, r, 128), xa.reshape(r, 128), xb.reshape(r, 128))

    prop = _make_prop(nb, npad)
    acc1a, acc1b = prop(src, dst, ua.reshape(npad), ub.reshape(npad))

    u2a, u2b = pl.pallas_call(
        _tc_b_body,
        out_shape=[_plane((r, 128))] * 2,
        in_specs=[_vspec()] * 5 + [_sspec()] * 3,
        out_specs=[_vspec()] * 2,
    )(acc1a.reshape(NC, r, 128), acc1b.reshape(NC, r, 128), dinv,
      ua, ub, W1, b1, W2)

    acc2a, acc2b = prop(src, dst, u2a.reshape(npad), u2b.reshape(npad))

    o0, o1 = pl.pallas_call(
        _tc_c_body,
        out_shape=[_plane((r, 128))] * 2,
        in_specs=[_vspec()] * 5 + [_sspec()],
        out_specs=[_vspec()] * 2,
    )(acc2a.reshape(NC, r, 128), acc2b.reshape(NC, r, 128), dinv,
      u2a, u2b, b2)

    return jnp.stack([o0.reshape(npad), o1.reshape(npad)], axis=1)[:n]


# speedup vs baseline: 198.0484x; 1.1162x over previous
import functools

import jax
import jax.numpy as jnp
from jax import lax
from jax.experimental import pallas as pl
from jax.experimental.pallas import tpu as pltpu
from jax.experimental.pallas import tpu_sc as plsc

NC = 2
NS = 16
NW = NC * NS
LANES = 16
B = 128
CB = 40


def _sc_mesh():
    return plsc.VectorSubcoreMesh(core_axis_name="c", subcore_axis_name="s")


def _zero_slice(zbuf, n):
    def body(i, _):
        zbuf[pl.ds(i * LANES, LANES)] = jnp.zeros((LANES,), jnp.float32)
        return _
    lax.fori_loop(0, n // LANES, body, None)


def _degree_kernel(nch, npad, dst_hbm, deg_out, dst_bufs, ones, zbuf, deg_sp, sems):
    s = lax.axis_index("s")
    c = lax.axis_index("c")
    wid = c * NS + s
    sl = npad // NS
    _zero_slice(zbuf, sl)
    pltpu.sync_copy(zbuf, deg_sp.at[pl.ds(s * sl, sl)])
    for i in range(B // LANES):
        ones[pl.ds(i * LANES, LANES)] = jnp.ones((LANES,), jnp.float32)
    plsc.subcore_barrier()

    nt = (nch + NW - 1) // NW

    def fire(dbuf, sem):
        def body(j, _):
            pltpu.async_copy(ones, deg_sp.at[dbuf.at[j]], sem, add=True)
            return _
        lax.fori_loop(0, CB, body, None)

    def drain(dbuf, sem):
        def body(j, _):
            pltpu.make_async_copy(ones, deg_sp.at[dbuf.at[j]], sem).wait()
            return _
        lax.fori_loop(0, CB, body, None)

    for t in range(nt + 1):
        p = t % 2
        q = (t - 1) % 2
        if t >= 1 and t - 1 < nt:
            ch1 = wid + NW * (t - 1)
            @pl.when(ch1 < nch)
            def _():
                drain(dst_bufs[q], sems[q])
        if t < nt:
            ch = wid + NW * t
            @pl.when(ch < nch)
            def _():
                pltpu.sync_copy(dst_hbm.at[pl.ds(ch * CB, CB)], dst_bufs[p])
                fire(dst_bufs[p], sems[p])

    plsc.subcore_barrier()
    pltpu.sync_copy(deg_sp.at[pl.ds(s * sl, sl)],
                    deg_out.at[c, pl.ds(s * sl, sl)])


def _prop_kernel(nch, npad, src_hbm, dst_hbm, ua_hbm, ub_hbm, acca_out, accb_out,
                 src_bufs, dst_bufs, ga_bufs, gb_bufs, zbuf,
                 ua_sp, ub_sp, aa_sp, ab_sp, gsems, ssems, stsems):
    s = lax.axis_index("s")
    c = lax.axis_index("c")
    wid = c * NS + s
    sl = npad // NS
    _zero_slice(zbuf, sl)
    pltpu.sync_copy(zbuf, aa_sp.at[pl.ds(s * sl, sl)])
    pltpu.sync_copy(zbuf, ab_sp.at[pl.ds(s * sl, sl)])
    pltpu.sync_copy(ua_hbm.at[pl.ds(s * sl, sl)], ua_sp.at[pl.ds(s * sl, sl)])
    pltpu.sync_copy(ub_hbm.at[pl.ds(s * sl, sl)], ub_sp.at[pl.ds(s * sl, sl)])
    plsc.subcore_barrier()

    nt = (nch + NW - 1) // NW

    def fire_gathers(sbuf, ga, gb, sem):
        def body(j, _):
            pltpu.async_copy(ua_sp.at[sbuf.at[j]], ga.at[j], sem)
            pltpu.async_copy(ub_sp.at[sbuf.at[j]], gb.at[j], sem)
            return _
        lax.fori_loop(0, CB, body, None)

    def drain_gathers(sbuf, ga, gb, sem):
        def body(j, _):
            pltpu.make_async_copy(ua_sp.at[sbuf.at[j]], ga.at[j], sem).wait()
            pltpu.make_async_copy(ub_sp.at[sbuf.at[j]], gb.at[j], sem).wait()
            return _
        lax.fori_loop(0, CB, body, None)

    def fire_scatters(dbuf, ga, gb, sem):
        def body(j, _):
            pltpu.async_copy(ga.at[j], aa_sp.at[dbuf.at[j]], sem, add=True)
            pltpu.async_copy(gb.at[j], ab_sp.at[dbuf.at[j]], sem, add=True)
            return _
        lax.fori_loop(0, CB, body, None)

    def drain_scatters(dbuf, ga, gb, sem):
        def body(j, _):
            pltpu.make_async_copy(ga.at[j], aa_sp.at[dbuf.at[j]], sem).wait()
            pltpu.make_async_copy(gb.at[j], ab_sp.at[dbuf.at[j]], sem).wait()
            return _
        lax.fori_loop(0, CB, body, None)

    def stage(hbm, buf, ch, sem):
        pltpu.async_copy(hbm.at[pl.ds(ch * CB, CB)], buf, sem)

    def wait_stage(hbm, buf, ch, sem):
        pltpu.make_async_copy(hbm.at[pl.ds(ch * CB, CB)], buf, sem).wait()

    @pl.when(wid < nch)
    def _():
        stage(src_hbm, src_bufs[0], wid, stsems[0])
        stage(dst_hbm, dst_bufs[0], wid, stsems[0])

    for t in range(nt + 2):
        p = t % 3
        q = (t - 1) % 3
        f = (t + 1) % 3
        if t >= 2 and t - 2 < nt:
            ch2 = wid + NW * (t - 2)
            @pl.when(ch2 < nch)
            def _():
                drain_scatters(dst_bufs[(t - 2) % 3], ga_bufs[(t - 2) % 3],
                               gb_bufs[(t - 2) % 3], ssems[(t - 2) % 3])
        if t < nt:
            ch = wid + NW * t
            @pl.when(ch < nch)
            def _():
                wait_stage(src_hbm, src_bufs[p], ch, stsems[p])
                wait_stage(dst_hbm, dst_bufs[p], ch, stsems[p])
                fire_gathers(src_bufs[p], ga_bufs[p], gb_bufs[p], gsems[p])
        if 1 <= t <= nt:
            ch1 = wid + NW * (t - 1)
            @pl.when(ch1 < nch)
            def _():
                drain_gathers(src_bufs[q], ga_bufs[q], gb_bufs[q], gsems[q])
                fire_scatters(dst_bufs[q], ga_bufs[q], gb_bufs[q], ssems[q])
        if t + 1 < nt:
            chn = wid + NW * (t + 1)
            @pl.when(chn < nch)
            def _():
                stage(src_hbm, src_bufs[f], chn, stsems[f])
                stage(dst_hbm, dst_bufs[f], chn, stsems[f])

    plsc.subcore_barrier()
    pltpu.sync_copy(aa_sp.at[pl.ds(s * sl, sl)], acca_out.at[c, pl.ds(s * sl, sl)])
    pltpu.sync_copy(ab_sp.at[pl.ds(s * sl, sl)], accb_out.at[c, pl.ds(s * sl, sl)])


def _make_degree(nb, npad):
    nch = nb // CB
    return pl.kernel(
        functools.partial(_degree_kernel, nch, npad),
        out_type=jax.ShapeDtypeStruct((NC, npad), jnp.float32),
        mesh=_sc_mesh(),
        scratch_types=[
            [pltpu.VMEM((CB, B), jnp.int32)] * 2,
            pltpu.VMEM((B,), jnp.float32),
            pltpu.VMEM((npad // NS,), jnp.float32),
            pltpu.MemorySpace.VMEM_SHARED((npad,), jnp.float32),
            [pltpu.SemaphoreType.DMA] * 2,
        ],
    )


def _make_prop(nb, npad):
    nch = nb // CB
    return pl.kernel(
        functools.partial(_prop_kernel, nch, npad),
        out_type=(jax.ShapeDtypeStruct((NC, npad), jnp.float32),
                  jax.ShapeDtypeStruct((NC, npad), jnp.float32)),
        mesh=_sc_mesh(),
        scratch_types=[
            [pltpu.VMEM((CB, B), jnp.int32)] * 3,
            [pltpu.VMEM((CB, B), jnp.int32)] * 3,
            [pltpu.VMEM((CB, B), jnp.float32)] * 3,
            [pltpu.VMEM((CB, B), jnp.float32)] * 3,
            pltpu.VMEM((npad // NS,), jnp.float32),
            pltpu.MemorySpace.VMEM_SHARED((npad,), jnp.float32),
            pltpu.MemorySpace.VMEM_SHARED((npad,), jnp.float32),
            pltpu.MemorySpace.VMEM_SHARED((npad,), jnp.float32),
            pltpu.MemorySpace.VMEM_SHARED((npad,), jnp.float32),
            [pltpu.SemaphoreType.DMA] * 3,
            [pltpu.SemaphoreType.DMA] * 3,
            [pltpu.SemaphoreType.DMA] * 3,
        ],
    )


def _tc_a_body(degp_ref, xa_ref, xb_ref, dinv_ref, ua_ref, ub_ref):
    deg = degp_ref[0] + degp_ref[1] + 1.0
    dinv = lax.rsqrt(deg)
    dinv_ref[...] = dinv
    ua_ref[...] = dinv * xa_ref[...]
    ub_ref[...] = dinv * xb_ref[...]


def _tc_b_body(acca_ref, accb_ref, dinv_ref, ua_ref, ub_ref,
               w1_ref, b1_ref, w2_ref, u2a_ref, u2b_ref):
    dinv = dinv_ref[...]
    pre0 = dinv * (acca_ref[0] + acca_ref[1] + ua_ref[...])
    pre1 = dinv * (accb_ref[0] + accb_ref[1] + ub_ref[...])
    y0 = jnp.zeros_like(pre0)
    y1 = jnp.zeros_like(pre0)
    for j in range(16):
        h = jnp.maximum(pre0 * w1_ref[0, j] + pre1 * w1_ref[1, j] + b1_ref[j], 0.0)
        y0 = y0 + h * w2_ref[j, 0]
        y1 = y1 + h * w2_ref[j, 1]
    u2a_ref[...] = dinv * y0
    u2b_ref[...] = dinv * y1


def _tc_c_body(acca_ref, accb_ref, dinv_ref, u2a_ref, u2b_ref, b2_ref,
               o0_ref, o1_ref):
    dinv = dinv_ref[...]
    z0 = dinv * (acca_ref[0] + acca_ref[1] + u2a_ref[...]) + b2_ref[0]
    z1 = dinv * (accb_ref[0] + accb_ref[1] + u2b_ref[...]) + b2_ref[1]
    m = jnp.maximum(z0, z1)
    ls = jnp.log(jnp.exp(z0 - m) + jnp.exp(z1 - m))
    o0_ref[...] = z0 - m - ls
    o1_ref[...] = z1 - m - ls


def _vspec():
    return pl.BlockSpec(memory_space=pltpu.MemorySpace.VMEM)


def _sspec():
    return pl.BlockSpec(memory_space=pltpu.MemorySpace.SMEM)


def _plane(shape):
    return jax.ShapeDtypeStruct(shape, jnp.float32)


def kernel(x1, edge_index1, x2, edge_index2, W1, b1, W2, b2):
    del x2, edge_index2
    n = x1.shape[0]
    e = edge_index1.shape[1]
    npad = ((n + 1023) // 1024) * 1024
    assert e % (B * CB) == 0
    nb = e // B
    r = npad // 128

    src = edge_index1[0].reshape(nb, B)
    dst = edge_index1[1].reshape(nb, B)
    xp = jnp.pad(x1, ((0, npad - n), (0, 0)))
    xa = xp[:, 0]
    xb = xp[:, 1]

    degp = _make_degree(nb, npad)(dst)

    dinv, ua, ub = pl.pallas_call(
        _tc_a_body,
        out_shape=[_plane((r, 128))] * 3,
        in_specs=[_vspec()] * 3,
        out_specs=[_vspec()] * 3,
    )(degp.reshape(NC, r, 128), xa.reshape(r, 128), xb.reshape(r, 128))

    prop = _make_prop(nb, npad)
    acc1a, acc1b = prop(src, dst, ua.reshape(npad), ub.reshape(npad))

    u2a, u2b = pl.pallas_call(
        _tc_b_body,
        out_shape=[_plane((r, 128))] * 2,
        in_specs=[_vspec()] * 5 + [_sspec()] * 3,
        out_specs=[_vspec()] * 2,
    )(acc1a.reshape(NC, r, 128), acc1b.reshape(NC, r, 128), dinv,
      ua, ub, W1, b1, W2)

    acc2a, acc2b = prop(src, dst, u2a.reshape(npad), u2b.reshape(npad))

    o0, o1 = pl.pallas_call(
        _tc_c_body,
        out_shape=[_plane((r, 128))] * 2,
        in_specs=[_vspec()] * 5 + [_sspec()],
        out_specs=[_vspec()] * 2,
    )(acc2a.reshape(NC, r, 128), acc2b.reshape(NC, r, 128), dinv,
      u2a, u2b, b2)

    return jnp.stack([o0.reshape(npad), o1.reshape(npad)], axis=1)[:n]


# speedup vs baseline: 205.4415x; 1.0373x over previous
import functools

import jax
import jax.numpy as jnp
from jax import lax
from jax.experimental import pallas as pl
from jax.experimental.pallas import tpu as pltpu
from jax.experimental.pallas import tpu_sc as plsc

NC = 2
NS = 16
NW = NC * NS
LANES = 16
B = 128
CB = 40


def _sc_mesh():
    return plsc.VectorSubcoreMesh(core_axis_name="c", subcore_axis_name="s")


def _zero_slice(zbuf, n):
    def body(i, _):
        zbuf[pl.ds(i * LANES, LANES)] = jnp.zeros((LANES,), jnp.float32)
        return _
    lax.fori_loop(0, n // LANES, body, None)


def _degree_kernel(nch, npad, dst_hbm, deg_out, dst_bufs, ones, zbuf, deg_sp, sems, stsems):
    s = lax.axis_index("s")
    c = lax.axis_index("c")
    wid = c * NS + s
    sl = npad // NS
    _zero_slice(zbuf, sl)
    pltpu.sync_copy(zbuf, deg_sp.at[pl.ds(s * sl, sl)])
    for i in range(B // LANES):
        ones[pl.ds(i * LANES, LANES)] = jnp.ones((LANES,), jnp.float32)
    plsc.subcore_barrier()

    nt = (nch + NW - 1) // NW

    def fire(dbuf, sem):
        def body(j, _):
            pltpu.async_copy(ones, deg_sp.at[dbuf.at[j]], sem, add=True)
            return _
        lax.fori_loop(0, CB, body, None)

    def drain(dbuf, sem):
        def body(j, _):
            pltpu.make_async_copy(ones, deg_sp.at[dbuf.at[j]], sem).wait()
            return _
        lax.fori_loop(0, CB, body, None)

    def stage(buf, ch, sem):
        pltpu.async_copy(dst_hbm.at[pl.ds(ch * CB, CB)], buf, sem)

    def wait_stage(buf, ch, sem):
        pltpu.make_async_copy(dst_hbm.at[pl.ds(ch * CB, CB)], buf, sem).wait()

    @pl.when(wid < nch)
    def _():
        stage(dst_bufs[0], wid, stsems[0])

    for t in range(nt + 1):
        p = t % 3
        f = (t + 1) % 3
        if t >= 1 and t - 1 < nt:
            ch1 = wid + NW * (t - 1)
            @pl.when(ch1 < nch)
            def _():
                drain(dst_bufs[(t - 1) % 3], sems[(t - 1) % 3])
        if t < nt:
            ch = wid + NW * t
            @pl.when(ch < nch)
            def _():
                wait_stage(dst_bufs[p], ch, stsems[p])
                fire(dst_bufs[p], sems[p])
        if t + 1 < nt:
            chn = wid + NW * (t + 1)
            @pl.when(chn < nch)
            def _():
                stage(dst_bufs[f], chn, stsems[f])

    plsc.subcore_barrier()
    pltpu.sync_copy(deg_sp.at[pl.ds(s * sl, sl)],
                    deg_out.at[c, pl.ds(s * sl, sl)])


def _prop_kernel(nch, npad, src_hbm, dst_hbm, ua_hbm, ub_hbm, acca_out, accb_out,
                 src_bufs, dst_bufs, ga_bufs, gb_bufs, zbuf,
                 ua_sp, ub_sp, aa_sp, ab_sp, gsems, ssems, stsems):
    s = lax.axis_index("s")
    c = lax.axis_index("c")
    wid = c * NS + s
    sl = npad // NS
    _zero_slice(zbuf, sl)
    pltpu.sync_copy(zbuf, aa_sp.at[pl.ds(s * sl, sl)])
    pltpu.sync_copy(zbuf, ab_sp.at[pl.ds(s * sl, sl)])
    pltpu.sync_copy(ua_hbm.at[pl.ds(s * sl, sl)], ua_sp.at[pl.ds(s * sl, sl)])
    pltpu.sync_copy(ub_hbm.at[pl.ds(s * sl, sl)], ub_sp.at[pl.ds(s * sl, sl)])
    plsc.subcore_barrier()

    nt = (nch + NW - 1) // NW

    def fire_gathers(sbuf, ga, gb, sem):
        def body(j, _):
            pltpu.async_copy(ua_sp.at[sbuf.at[j]], ga.at[j], sem)
            pltpu.async_copy(ub_sp.at[sbuf.at[j]], gb.at[j], sem)
            return _
        lax.fori_loop(0, CB, body, None)

    def drain_gathers(sbuf, ga, gb, sem):
        def body(j, _):
            pltpu.make_async_copy(ua_sp.at[sbuf.at[j]], ga.at[j], sem).wait()
            pltpu.make_async_copy(ub_sp.at[sbuf.at[j]], gb.at[j], sem).wait()
            return _
        lax.fori_loop(0, CB, body, None)

    def fire_scatters(dbuf, ga, gb, sem):
        def body(j, _):
            pltpu.async_copy(ga.at[j], aa_sp.at[dbuf.at[j]], sem, add=True)
            pltpu.async_copy(gb.at[j], ab_sp.at[dbuf.at[j]], sem, add=True)
            return _
        lax.fori_loop(0, CB, body, None)

    def drain_scatters(dbuf, ga, gb, sem):
        def body(j, _):
            pltpu.make_async_copy(ga.at[j], aa_sp.at[dbuf.at[j]], sem).wait()
            pltpu.make_async_copy(gb.at[j], ab_sp.at[dbuf.at[j]], sem).wait()
            return _
        lax.fori_loop(0, CB, body, None)

    def stage(hbm, buf, ch, sem):
        pltpu.async_copy(hbm.at[pl.ds(ch * CB, CB)], buf, sem)

    def wait_stage(hbm, buf, ch, sem):
        pltpu.make_async_copy(hbm.at[pl.ds(ch * CB, CB)], buf, sem).wait()

    @pl.when(wid < nch)
    def _():
        stage(src_hbm, src_bufs[0], wid, stsems[0])
        stage(dst_hbm, dst_bufs[0], wid, stsems[0])

    for t in range(nt + 2):
        p = t % 3
        q = (t - 1) % 3
        f = (t + 1) % 3
        if t >= 2 and t - 2 < nt:
            ch2 = wid + NW * (t - 2)
            @pl.when(ch2 < nch)
            def _():
                drain_scatters(dst_bufs[(t - 2) % 3], ga_bufs[(t - 2) % 3],
                               gb_bufs[(t - 2) % 3], ssems[(t - 2) % 3])
        if t < nt:
            ch = wid + NW * t
            @pl.when(ch < nch)
            def _():
                wait_stage(src_hbm, src_bufs[p], ch, stsems[p])
                wait_stage(dst_hbm, dst_bufs[p], ch, stsems[p])
                fire_gathers(src_bufs[p], ga_bufs[p], gb_bufs[p], gsems[p])
        if 1 <= t <= nt:
            ch1 = wid + NW * (t - 1)
            @pl.when(ch1 < nch)
            def _():
                drain_gathers(src_bufs[q], ga_bufs[q], gb_bufs[q], gsems[q])
                fire_scatters(dst_bufs[q], ga_bufs[q], gb_bufs[q], ssems[q])
        if t + 1 < nt:
            chn = wid + NW * (t + 1)
            @pl.when(chn < nch)
            def _():
                stage(src_hbm, src_bufs[f], chn, stsems[f])
                stage(dst_hbm, dst_bufs[f], chn, stsems[f])

    plsc.subcore_barrier()
    pltpu.sync_copy(aa_sp.at[pl.ds(s * sl, sl)], acca_out.at[c, pl.ds(s * sl, sl)])
    pltpu.sync_copy(ab_sp.at[pl.ds(s * sl, sl)], accb_out.at[c, pl.ds(s * sl, sl)])


def _make_degree(nb, npad):
    nch = nb // CB
    return pl.kernel(
        functools.partial(_degree_kernel, nch, npad),
        out_type=jax.ShapeDtypeStruct((NC, npad), jnp.float32),
        mesh=_sc_mesh(),
        scratch_types=[
            [pltpu.VMEM((CB, B), jnp.int32)] * 3,
            pltpu.VMEM((B,), jnp.float32),
            pltpu.VMEM((npad // NS,), jnp.float32),
            pltpu.MemorySpace.VMEM_SHARED((npad,), jnp.float32),
            [pltpu.SemaphoreType.DMA] * 3,
            [pltpu.SemaphoreType.DMA] * 3,
        ],
    )


def _make_prop(nb, npad):
    nch = nb // CB
    return pl.kernel(
        functools.partial(_prop_kernel, nch, npad),
        out_type=(jax.ShapeDtypeStruct((NC, npad), jnp.float32),
                  jax.ShapeDtypeStruct((NC, npad), jnp.float32)),
        mesh=_sc_mesh(),
        scratch_types=[
            [pltpu.VMEM((CB, B), jnp.int32)] * 3,
            [pltpu.VMEM((CB, B), jnp.int32)] * 3,
            [pltpu.VMEM((CB, B), jnp.float32)] * 3,
            [pltpu.VMEM((CB, B), jnp.float32)] * 3,
            pltpu.VMEM((npad // NS,), jnp.float32),
            pltpu.MemorySpace.VMEM_SHARED((npad,), jnp.float32),
            pltpu.MemorySpace.VMEM_SHARED((npad,), jnp.float32),
            pltpu.MemorySpace.VMEM_SHARED((npad,), jnp.float32),
            pltpu.MemorySpace.VMEM_SHARED((npad,), jnp.float32),
            [pltpu.SemaphoreType.DMA] * 3,
            [pltpu.SemaphoreType.DMA] * 3,
            [pltpu.SemaphoreType.DMA] * 3,
        ],
    )


def _tc_a_body(degp_ref, xa_ref, xb_ref, dinv_ref, ua_ref, ub_ref):
    deg = degp_ref[0] + degp_ref[1] + 1.0
    dinv = lax.rsqrt(deg)
    dinv_ref[...] = dinv
    ua_ref[...] = dinv * xa_ref[...]
    ub_ref[...] = dinv * xb_ref[...]


def _tc_b_body(acca_ref, accb_ref, dinv_ref, ua_ref, ub_ref,
               w1_ref, b1_ref, w2_ref, u2a_ref, u2b_ref):
    dinv = dinv_ref[...]
    pre0 = dinv * (acca_ref[0] + acca_ref[1] + ua_ref[...])
    pre1 = dinv * (accb_ref[0] + accb_ref[1] + ub_ref[...])
    y0 = jnp.zeros_like(pre0)
    y1 = jnp.zeros_like(pre0)
    for j in range(16):
        h = jnp.maximum(pre0 * w1_ref[0, j] + pre1 * w1_ref[1, j] + b1_ref[j], 0.0)
        y0 = y0 + h * w2_ref[j, 0]
        y1 = y1 + h * w2_ref[j, 1]
    u2a_ref[...] = dinv * y0
    u2b_ref[...] = dinv * y1


def _tc_c_body(acca_ref, accb_ref, dinv_ref, u2a_ref, u2b_ref, b2_ref,
               o0_ref, o1_ref):
    dinv = dinv_ref[...]
    z0 = dinv * (acca_ref[0] + acca_ref[1] + u2a_ref[...]) + b2_ref[0]
    z1 = dinv * (accb_ref[0] + accb_ref[1] + u2b_ref[...]) + b2_ref[1]
    m = jnp.maximum(z0, z1)
    ls = jnp.log(jnp.exp(z0 - m) + jnp.exp(z1 - m))
    o0_ref[...] = z0 - m - ls
    o1_ref[...] = z1 - m - ls


def _vspec():
    return pl.BlockSpec(memory_space=pltpu.MemorySpace.VMEM)


def _sspec():
    return pl.BlockSpec(memory_space=pltpu.MemorySpace.SMEM)


def _plane(shape):
    return jax.ShapeDtypeStruct(shape, jnp.float32)


def kernel(x1, edge_index1, x2, edge_index2, W1, b1, W2, b2):
    del x2, edge_index2
    n = x1.shape[0]
    e = edge_index1.shape[1]
    npad = ((n + 1023) // 1024) * 1024
    assert e % (B * CB) == 0
    nb = e // B
    r = npad // 128

    src = edge_index1[0].reshape(nb, B)
    dst = edge_index1[1].reshape(nb, B)
    xp = jnp.pad(x1, ((0, npad - n), (0, 0)))
    xa = xp[:, 0]
    xb = xp[:, 1]

    degp = _make_degree(nb, npad)(dst)

    dinv, ua, ub = pl.pallas_call(
        _tc_a_body,
        out_shape=[_plane((r, 128))] * 3,
        in_specs=[_vspec()] * 3,
        out_specs=[_vspec()] * 3,
    )(degp.reshape(NC, r, 128), xa.reshape(r, 128), xb.reshape(r, 128))

    prop = _make_prop(nb, npad)
    acc1a, acc1b = prop(src, dst, ua.reshape(npad), ub.reshape(npad))

    u2a, u2b = pl.pallas_call(
        _tc_b_body,
        out_shape=[_plane((r, 128))] * 2,
        in_specs=[_vspec()] * 5 + [_sspec()] * 3,
        out_specs=[_vspec()] * 2,
    )(acc1a.reshape(NC, r, 128), acc1b.reshape(NC, r, 128), dinv,
      ua, ub, W1, b1, W2)

    acc2a, acc2b = prop(src, dst, u2a.reshape(npad), u2b.reshape(npad))

    o0, o1 = pl.pallas_call(
        _tc_c_body,
        out_shape=[_plane((r, 128))] * 2,
        in_specs=[_vspec()] * 5 + [_sspec()],
        out_specs=[_vspec()] * 2,
    )(acc2a.reshape(NC, r, 128), acc2b.reshape(NC, r, 128), dinv,
      u2a, u2b, b2)

    return jnp.stack([o0.reshape(npad), o1.reshape(npad)], axis=1)[:n]
